# baseline (speedup 1.0000x reference)
import functools

import jax
import jax.numpy as jnp
from jax import lax
from jax.experimental import pallas as pl
from jax.experimental.pallas import tpu as pltpu

_ANCHORS_W = (10.0, 16.0, 33.0)
_ANCHORS_H = (13.0, 30.0, 23.0)
_NA = 3
_NC = 80
_C = _NC + 5


def _yolo_body(stride_ref, x_ref, o_ref, *, g):
    i = pl.program_id(0)
    stride = stride_ref[0, 0]
    t = x_ref[0]
    B = t.shape[1]
    sig = jax.nn.sigmoid(t)
    o_ref[0] = sig
    gx = lax.broadcasted_iota(jnp.int32, (g, B, 1), 0).astype(jnp.float32)
    gy = i.astype(jnp.float32)
    gxy = jnp.concatenate([gx, jnp.full((g, B, 1), 0.0) + gy], axis=2)
    for a in range(_NA):
        base = a * _C
        o_ref[0, :, :, base:base + 2] = (sig[:, :, base:base + 2] + gxy) * stride
        e = jnp.exp(t[:, :, base + 2:base + 4])
        lane = lax.broadcasted_iota(jnp.int32, (1, 1, 2), 2)
        wh = jnp.where(lane == 0, _ANCHORS_W[a], _ANCHORS_H[a])
        o_ref[0, :, :, base + 2:base + 4] = e * wh


def kernel(x, img_dim):
    B = x.shape[0]
    g = x.shape[2]
    n = g * g
    stride = (jnp.asarray(img_dim, jnp.float32) / g).reshape(1, 1)
    xt = jnp.transpose(x, (2, 3, 0, 1))
    y = pl.pallas_call(
        functools.partial(_yolo_body, g=g),
        grid=(g,),
        in_specs=[
            pl.BlockSpec((1, 1), lambda i: (0, 0)),
            pl.BlockSpec((1, g, B, _NA * _C), lambda i: (i, 0, 0, 0)),
        ],
        out_specs=pl.BlockSpec((1, g, B, _NA * _C), lambda i: (i, 0, 0, 0)),
        out_shape=jax.ShapeDtypeStruct((g, g, B, _NA * _C), jnp.float32),
        compiler_params=pltpu.CompilerParams(
            dimension_semantics=("arbitrary",),
        ),
    )(stride, xt)
    y5 = y.reshape(g, g, B, _NA, _C)
    return y5.transpose(2, 3, 0, 1, 4).reshape(B, _NA * n, _C)


# speedup vs baseline: 2.0333x; 2.0333x over previous
import functools

import jax
import jax.numpy as jnp
from jax import lax
from jax.experimental import pallas as pl
from jax.experimental.pallas import tpu as pltpu

_ANCHORS_W = (10.0, 16.0, 33.0)
_ANCHORS_H = (13.0, 30.0, 23.0)
_NA = 3
_NC = 80
_C = _NC + 5


def _yolo_body(stride_ref, x0_ref, x1_ref, x2_ref, o_ref, *, g):
    c = pl.program_id(0)
    stride = stride_ref[0, 0]
    B = x0_ref.shape[0]
    n = g * g
    f0 = x0_ref[...].reshape(B, n)
    f1 = x1_ref[...].reshape(B, n)
    f2 = x2_ref[...].reshape(B, n)
    t = jnp.concatenate([f0, f1, f2], axis=1)
    sig = jax.nn.sigmoid(t)

    q = lax.broadcasted_iota(jnp.int32, (B, _NA * n), 1)
    cell = q % n

    @pl.when(c == 0)
    def _():
        gx = (cell % g).astype(jnp.float32)
        o_ref[0] = (sig + gx) * stride

    @pl.when(c == 1)
    def _():
        gy = (cell // g).astype(jnp.float32)
        o_ref[0] = (sig + gy) * stride

    @pl.when(c == 2)
    def _():
        aw = jnp.where(q < n, _ANCHORS_W[0], jnp.where(q < 2 * n, _ANCHORS_W[1], _ANCHORS_W[2]))
        o_ref[0] = jnp.exp(t) * aw

    @pl.when(c == 3)
    def _():
        ah = jnp.where(q < n, _ANCHORS_H[0], jnp.where(q < 2 * n, _ANCHORS_H[1], _ANCHORS_H[2]))
        o_ref[0] = jnp.exp(t) * ah

    @pl.when(c >= 4)
    def _():
        o_ref[0] = sig


def kernel(x, img_dim):
    B = x.shape[0]
    g = x.shape[2]
    n = g * g
    stride = (jnp.asarray(img_dim, jnp.float32) / g).reshape(1, 1)
    op = pl.pallas_call(
        functools.partial(_yolo_body, g=g),
        grid=(_C,),
        in_specs=[
            pl.BlockSpec((1, 1), lambda c: (0, 0)),
            pl.BlockSpec((B, 1, g, g), lambda c: (0, c, 0, 0)),
            pl.BlockSpec((B, 1, g, g), lambda c: (0, c + _C, 0, 0)),
            pl.BlockSpec((B, 1, g, g), lambda c: (0, c + 2 * _C, 0, 0)),
        ],
        out_specs=pl.BlockSpec((1, B, _NA * n), lambda c: (c, 0, 0)),
        out_shape=jax.ShapeDtypeStruct((_C, B, _NA * n), jnp.float32),
        compiler_params=pltpu.CompilerParams(
            dimension_semantics=("arbitrary",),
        ),
    )(stride, x, x, x)
    return jnp.transpose(op, (1, 2, 0))
